# Initial kernel scaffold; baseline (speedup 1.0000x reference)
import jax
import jax.numpy as jnp
from jax.experimental import pallas as pl


def kernel(x, edge_index, edge_attr, atom_emb, bond_emb, gat_W, att_src, att_dst, gat_We, att_edge, gat_bias, ln_g, ln_b, eu_W, eu_b):
    raise NotImplementedError("write your pallas kernel here")


# speedup vs baseline: 11.4580x; 11.4580x over previous
import functools

import jax
import jax.numpy as jnp
from jax import lax
from jax.experimental import pallas as pl
from jax.experimental.pallas import tpu as pltpu
from jax.experimental.pallas import tpu_sc as plsc

N = 50000
E = 800000
D = 64
H = 4
C = 16
L = 4
NC = 2
NS = 16
NP = 51200
NTW = NP // NS

EH = E // NC
ET_HALF = EH // NS
ET_FULL = E // NS

_SC_PARAMS = pltpu.CompilerParams(use_tc_tiling_on_sc=False)


@functools.lru_cache(maxsize=None)
def _sc_mesh():
    return plsc.VectorSubcoreMesh(core_axis_name="c", subcore_axis_name="s",
                                  num_cores=NC, num_subcores=NS)


def _enc_node_body(x_ref, wn_ref, o_ref):
    xb = x_ref[...]
    cols = []
    io = lax.broadcasted_iota(jnp.int32, (xb.shape[0], 16), 1)
    for i in range(9):
        cols.append((xb[:, i:i + 1] == io).astype(jnp.float32))
    oh = jnp.concatenate(cols, axis=1)
    o_ref[...] = jnp.dot(oh, wn_ref[...], preferred_element_type=jnp.float32)


def _enc_edge_body(a_ref, wb_ref, o_ref):
    ab = a_ref[...]
    io = lax.broadcasted_iota(jnp.int32, (ab.shape[0], 8), 1)
    cols = [(ab[:, i:i + 1] == io).astype(jnp.float32) for i in range(3)]
    oh = jnp.concatenate(cols, axis=1)
    o_ref[...] = jnp.dot(oh, wb_ref[...], preferred_element_type=jnp.float32)


def _node_pre_body(nd_ref, w_ref, asr_ref, adr_ref, hst_ref, as_ref, ad_ref):
    h = jnp.dot(nd_ref[...], w_ref[...], preferred_element_type=jnp.float32)
    asr = asr_ref[...]
    adr = adr_ref[...]
    a_s = []
    a_d = []
    for hd in range(H):
        hh = h[:, hd * C:(hd + 1) * C]
        a_s.append(jnp.sum(hh * asr[:, hd * C:(hd + 1) * C], axis=1, keepdims=True))
        a_d.append(jnp.sum(hh * adr[:, hd * C:(hd + 1) * C], axis=1, keepdims=True))
    zpad = jnp.zeros((h.shape[0], 12), dtype=jnp.float32)
    as_ref[...] = jnp.concatenate(a_s + [zpad], axis=1)
    ad_ref[...] = jnp.concatenate(a_d + [zpad], axis=1)
    hst_ref[...] = jnp.stack([h[:, :32], h[:, 32:]], axis=0)


def _edge_pre_body(ed_ref, we_ref, aer_ref, est_ref, ae_ref):
    e = jnp.dot(ed_ref[...], we_ref[...], preferred_element_type=jnp.float32)
    aer = aer_ref[...]
    a_e = []
    for hd in range(H):
        ee = e[:, hd * C:(hd + 1) * C]
        a_e.append(jnp.sum(ee * aer[:, hd * C:(hd + 1) * C], axis=1, keepdims=True))
    zpad = jnp.zeros((e.shape[0], 12), dtype=jnp.float32)
    ae_ref[...] = jnp.concatenate(a_e + [zpad], axis=1)
    est_ref[...] = jnp.stack([e[:, :32], e[:, 32:]], axis=0)


def _ex_body(gs_ref, gd_ref, ae_ref, ex_ref):
    al = gs_ref[...] + gd_ref[...] + ae_ref[...]
    al = jnp.maximum(al, 0.2 * al)
    ex_ref[...] = jnp.exp(al)


def _rden_body(p0_ref, p1_ref, o_ref):
    o_ref[...] = 1.0 / (p0_ref[...] + p1_ref[...] + 1e-16)


def _msg_body(hs_ref, es_ref, ex_ref, rd_ref, msg_ref):
    attn = ex_ref[...] * rd_ref[...]
    he = hs_ref[...] + es_ref[...]
    ones = jnp.ones((1, C), dtype=jnp.float32)
    halves = []
    for f in range(2):
        aa = jnp.concatenate(
            [attn[:, 2 * f + j:2 * f + j + 1] * ones for j in range(2)], axis=1)
        halves.append(he[f] * aa)
    msg_ref[...] = jnp.stack(halves, axis=0)


def _node_post_body(out_ref, nd_ref, bias_ref, g_ref, b_ref, wb_ref, wc_ref,
                    nn_ref, nb_ref, nc_ref):
    ob = out_ref[...]
    out = jnp.concatenate([ob[0], ob[1]], axis=1) + bias_ref[...]
    mu = jnp.mean(out, axis=1, keepdims=True)
    xc = out - mu
    var = jnp.mean(xc * xc, axis=1, keepdims=True)
    conv = g_ref[...] * xc * lax.rsqrt(var + 1e-5) + b_ref[...]
    nn = jnp.maximum(conv, 0.0) + nd_ref[...]
    nn_ref[...] = nn
    nb = jnp.dot(nn, wb_ref[...], preferred_element_type=jnp.float32)
    ncv = jnp.dot(nn, wc_ref[...], preferred_element_type=jnp.float32)
    nb_ref[...] = jnp.stack([nb[:, :32], nb[:, 32:]], axis=0)
    nc_ref[...] = jnp.stack([ncv[:, :32], ncv[:, 32:]], axis=0)


def _edge_upd_body(ed_ref, nb_ref, nc_ref, wa_ref, eb_ref, o_ref):
    nb = nb_ref[...]
    nc = nc_ref[...]
    sd64 = jnp.concatenate([nb[0] + nc[0], nb[1] + nc[1]], axis=1)
    up = jnp.dot(ed_ref[...], wa_ref[...], preferred_element_type=jnp.float32)
    o_ref[...] = jnp.maximum(ed_ref[...] + up + sd64 + eb_ref[...], 0.0)


def _full(shape):
    return pl.BlockSpec(shape, lambda i: tuple(0 for _ in shape))


SB = 128
TH = ET_HALF - (ET_HALF // SB) * SB
TF = ET_FULL - (ET_FULL // SB) * SB
NBW = NTW // SB


def _sc_gather_ad_body(src_hbm, dst_hbm, ast_hbm, adt_hbm, gs_hbm, gd_hbm,
                       sv, dv, gsv, gdv, svt, dvt, gsvt, gdvt, sem):
    cid = lax.axis_index("c")
    sid = lax.axis_index("s")
    tile_base = cid * EH + sid * ET_HALF

    def do(base, n, a_sv, a_dv, a_gsv, a_gdv):
        pltpu.sync_copy(src_hbm.at[pl.ds(base, n)], a_sv)
        pltpu.sync_copy(dst_hbm.at[pl.ds(base, n)], a_dv)
        pltpu.async_copy(ast_hbm.at[a_sv], a_gsv, sem).wait()
        pltpu.async_copy(adt_hbm.at[a_dv], a_gdv, sem).wait()
        pltpu.sync_copy(a_gsv, gs_hbm.at[pl.ds(base, n)])
        pltpu.sync_copy(a_gdv, gd_hbm.at[pl.ds(base, n)])

    pl.loop(0, ET_HALF // SB)(
        lambda k: do(tile_base + k * SB, SB, sv, dv, gsv, gdv))
    do(tile_base + (ET_HALF // SB) * SB, TH, svt, dvt, gsvt, gdvt)


def _sc_den_body(dst_hbm, ex_hbm, z4_hbm, den_hbm,
                 dv, exv, dvt, exvt, zb, den_sp):
    cid = lax.axis_index("c")
    sid = lax.axis_index("s")
    w0 = sid * NTW
    pltpu.sync_copy(z4_hbm.at[pl.ds(0, SB)], zb)
    pl.loop(0, NBW)(
        lambda j: pltpu.sync_copy(zb, den_sp.at[pl.ds(w0 + j * SB, SB)]))
    plsc.subcore_barrier()
    tile_base = cid * EH + sid * ET_HALF

    def do(base, n, a_dv, a_exv):
        pltpu.sync_copy(dst_hbm.at[pl.ds(base, n)], a_dv)
        pltpu.sync_copy(ex_hbm.at[pl.ds(base, n)], a_exv)
        pltpu.sync_copy(a_exv, den_sp.at[a_dv], add=True)

    pl.loop(0, ET_HALF // SB)(lambda k: do(tile_base + k * SB, SB, dv, exv))
    do(tile_base + (ET_HALF // SB) * SB, TH, dvt, exvt)
    plsc.subcore_barrier()

    def wb(j):
        pltpu.sync_copy(den_sp.at[pl.ds(w0 + j * SB, SB)], zb)
        pltpu.sync_copy(zb, den_hbm.at[pl.ds(cid * NP + w0 + j * SB, SB)])

    pl.loop(0, NBW)(wb)


def _sc_gather_h_body(src_hbm, dst_hbm, h_hbm, rden_hbm, hs_hbm, rd_hbm,
                      sv, dv, iv, hv, rdv, svt, dvt, ivt, hvt, rdvt, sem):
    cid = lax.axis_index("c")
    sid = lax.axis_index("s")
    tile_base = sid * ET_FULL

    def do(base, n, a_sv, a_dv, a_iv, a_hv, a_rdv):
        pltpu.sync_copy(src_hbm.at[pl.ds(base, n)], a_sv)
        for r in range(n // 16):
            a_iv[pl.ds(r * 16, 16)] = a_sv[pl.ds(r * 16, 16)] + cid * N
        pltpu.async_copy(h_hbm.at[a_iv], a_hv, sem).wait()
        pltpu.sync_copy(a_hv, hs_hbm.at[pl.ds(cid * E + base, n)])

        @pl.when(cid == 0)
        def _():
            pltpu.sync_copy(dst_hbm.at[pl.ds(base, n)], a_dv)
            pltpu.async_copy(rden_hbm.at[a_dv], a_rdv, sem).wait()
            pltpu.sync_copy(a_rdv, rd_hbm.at[pl.ds(base, n)])

    pl.loop(0, ET_FULL // SB)(
        lambda k: do(tile_base + k * SB, SB, sv, dv, iv, hv, rdv))
    do(tile_base + (ET_FULL // SB) * SB, TF, svt, dvt, ivt, hvt, rdvt)


def _sc_scatter_out_body(dst_hbm, msg_hbm, z32_hbm, out_hbm,
                         dv, mv, dvt, mvt, zb, out_sp):
    cid = lax.axis_index("c")
    sid = lax.axis_index("s")
    w0 = sid * NTW
    pltpu.sync_copy(z32_hbm.at[pl.ds(0, SB)], zb)
    pl.loop(0, NBW)(
        lambda j: pltpu.sync_copy(zb, out_sp.at[pl.ds(w0 + j * SB, SB)]))
    plsc.subcore_barrier()
    tile_base = sid * ET_FULL

    def do(base, n, a_dv, a_mv):
        pltpu.sync_copy(dst_hbm.at[pl.ds(base, n)], a_dv)
        pltpu.sync_copy(msg_hbm.at[pl.ds(cid * E + base, n)], a_mv)
        pltpu.sync_copy(a_mv, out_sp.at[a_dv], add=True)

    pl.loop(0, ET_FULL // SB)(lambda k: do(tile_base + k * SB, SB, dv, mv))
    do(tile_base + (ET_FULL // SB) * SB, TF, dvt, mvt)
    plsc.subcore_barrier()

    def wb(j):
        pltpu.sync_copy(out_sp.at[pl.ds(w0 + j * SB, SB)], zb)
        pltpu.sync_copy(zb, out_hbm.at[pl.ds(cid * NP + w0 + j * SB, SB)])

    pl.loop(0, NBW)(wb)


def _sc_gather_nbnc_body(src_hbm, dst_hbm, nb_hbm, nc_hbm, nbg_hbm, ncg_hbm,
                         sv, dv, isv, idv, bv, cv,
                         svt, dvt, isvt, idvt, bvt, cvt, sem):
    cid = lax.axis_index("c")
    sid = lax.axis_index("s")
    tile_base = sid * ET_FULL

    def do(base, n, a_sv, a_dv, a_isv, a_idv, a_bv, a_cv):
        pltpu.sync_copy(src_hbm.at[pl.ds(base, n)], a_sv)
        pltpu.sync_copy(dst_hbm.at[pl.ds(base, n)], a_dv)
        for r in range(n // 16):
            a_isv[pl.ds(r * 16, 16)] = a_sv[pl.ds(r * 16, 16)] + cid * N
            a_idv[pl.ds(r * 16, 16)] = a_dv[pl.ds(r * 16, 16)] + cid * N
        pltpu.async_copy(nb_hbm.at[a_isv], a_bv, sem).wait()
        pltpu.async_copy(nc_hbm.at[a_idv], a_cv, sem).wait()
        pltpu.sync_copy(a_bv, nbg_hbm.at[pl.ds(cid * E + base, n)])
        pltpu.sync_copy(a_cv, ncg_hbm.at[pl.ds(cid * E + base, n)])

    pl.loop(0, ET_FULL // SB)(
        lambda k: do(tile_base + k * SB, SB, sv, dv, isv, idv, bv, cv))
    do(tile_base + (ET_FULL // SB) * SB, TF, svt, dvt, isvt, idvt, bvt, cvt)


def _tc(body, grid, in_specs, out_specs, out_shapes):
    return pl.pallas_call(body, grid=grid, in_specs=in_specs,
                          out_specs=out_specs, out_shape=out_shapes)


def kernel(x, edge_index, edge_attr, atom_emb, bond_emb, gat_W, att_src,
           att_dst, gat_We, att_edge, gat_bias, ln_g, ln_b, eu_W, eu_b):
    f32 = jnp.float32
    src = edge_index[0]
    dst = edge_index[1]

    wn = atom_emb[:, :16, :].reshape(144, D)
    bn = 2000
    node = _tc(_enc_node_body, (N // bn,),
               [pl.BlockSpec((bn, 9), lambda i: (i, 0)), _full((144, D))],
               pl.BlockSpec((bn, D), lambda i: (i, 0)),
               jax.ShapeDtypeStruct((N, D), f32))(x, wn)

    wb = bond_emb[:, :8, :].reshape(24, D)
    be = 8000
    edge = _tc(_enc_edge_body, (E // be,),
               [pl.BlockSpec((be, 3), lambda i: (i, 0)), _full((24, D))],
               pl.BlockSpec((be, D), lambda i: (i, 0)),
               jax.ShapeDtypeStruct((E, D), f32))(edge_attr, wb)

    z16 = jnp.zeros((NP, 16), f32)
    z32 = jnp.zeros((NP, 32), f32)

    for l in range(L):
        h_st, as_t, ad_t = _tc(
            _node_pre_body, (N // bn,),
            [pl.BlockSpec((bn, D), lambda i: (i, 0)), _full((D, D)),
             _full((1, D)), _full((1, D))],
            [pl.BlockSpec((2, bn, 32), lambda i: (0, i, 0)),
             pl.BlockSpec((bn, 16), lambda i: (i, 0)),
             pl.BlockSpec((bn, 16), lambda i: (i, 0))],
            [jax.ShapeDtypeStruct((2, N, 32), f32),
             jax.ShapeDtypeStruct((N, 16), f32),
             jax.ShapeDtypeStruct((N, 16), f32)],
        )(node, gat_W[l], att_src[l].reshape(1, D), att_dst[l].reshape(1, D))
        h_flat = h_st.reshape(2 * N, 32)

        e_st, ae_t = _tc(
            _edge_pre_body, (E // be,),
            [pl.BlockSpec((be, D), lambda i: (i, 0)), _full((D, D)),
             _full((1, D))],
            [pl.BlockSpec((2, be, 32), lambda i: (0, i, 0)),
             pl.BlockSpec((be, 16), lambda i: (i, 0))],
            [jax.ShapeDtypeStruct((2, E, 32), f32),
             jax.ShapeDtypeStruct((E, 16), f32)],
        )(edge, gat_We[l], att_edge[l].reshape(1, D))
        e_flat = e_st.reshape(2 * E, 32)

        gs, gd = pl.kernel(
            _sc_gather_ad_body,
            out_type=[jax.ShapeDtypeStruct((E, 16), f32),
                      jax.ShapeDtypeStruct((E, 16), f32)],
            mesh=_sc_mesh(),
            compiler_params=_SC_PARAMS,
            scratch_types=[pltpu.VMEM((SB,), jnp.int32),
                           pltpu.VMEM((SB,), jnp.int32),
                           pltpu.VMEM((SB, 16), f32),
                           pltpu.VMEM((SB, 16), f32),
                           pltpu.VMEM((TH,), jnp.int32),
                           pltpu.VMEM((TH,), jnp.int32),
                           pltpu.VMEM((TH, 16), f32),
                           pltpu.VMEM((TH, 16), f32),
                           pltpu.SemaphoreType.DMA],
        )(src, dst, as_t, ad_t)

        rr = E // 8
        bx = 10000
        ex = _tc(_ex_body, (rr // bx,),
                 [pl.BlockSpec((bx, 128), lambda i: (i, 0))] * 3,
                 pl.BlockSpec((bx, 128), lambda i: (i, 0)),
                 jax.ShapeDtypeStruct((rr, 128), f32))(
            gs.reshape(rr, 128), gd.reshape(rr, 128),
            ae_t.reshape(rr, 128)).reshape(E, 16)

        den2 = pl.kernel(
            _sc_den_body,
            out_type=jax.ShapeDtypeStruct((2 * NP, 16), f32),
            mesh=_sc_mesh(),
            compiler_params=_SC_PARAMS,
            scratch_types=[pltpu.VMEM((SB,), jnp.int32),
                           pltpu.VMEM((SB, 16), f32),
                           pltpu.VMEM((TH,), jnp.int32),
                           pltpu.VMEM((TH, 16), f32),
                           pltpu.VMEM((SB, 16), f32),
                           pltpu.VMEM_SHARED((NP, 16), f32)],
        )(dst, ex, z16)

        brd = 6250
        rden = _tc(_rden_body, (1,),
                   [pl.BlockSpec((brd, 128), lambda i: (i, 0))] * 2,
                   pl.BlockSpec((brd, 128), lambda i: (i, 0)),
                   jax.ShapeDtypeStruct((brd, 128), f32))(
            den2[:N].reshape(brd, 128),
            den2[NP:NP + N].reshape(brd, 128)).reshape(N, 16)

        hsrc, rdsts = pl.kernel(
            _sc_gather_h_body,
            out_type=[jax.ShapeDtypeStruct((2 * E, 32), f32),
                      jax.ShapeDtypeStruct((E, 16), f32)],
            mesh=_sc_mesh(),
            compiler_params=_SC_PARAMS,
            scratch_types=[pltpu.VMEM((SB,), jnp.int32),
                           pltpu.VMEM((SB,), jnp.int32),
                           pltpu.VMEM((SB,), jnp.int32),
                           pltpu.VMEM((SB, 32), f32),
                           pltpu.VMEM((SB, 16), f32),
                           pltpu.VMEM((TF,), jnp.int32),
                           pltpu.VMEM((TF,), jnp.int32),
                           pltpu.VMEM((TF,), jnp.int32),
                           pltpu.VMEM((TF, 32), f32),
                           pltpu.VMEM((TF, 16), f32),
                           pltpu.SemaphoreType.DMA],
        )(src, dst, h_flat, rden)

        bm = 2000
        msg = _tc(
            _msg_body, (E // bm,),
            [pl.BlockSpec((2, bm, 32), lambda i: (0, i, 0)),
             pl.BlockSpec((2, bm, 32), lambda i: (0, i, 0)),
             pl.BlockSpec((bm, 16), lambda i: (i, 0)),
             pl.BlockSpec((bm, 16), lambda i: (i, 0))],
            pl.BlockSpec((2, bm, 32), lambda i: (0, i, 0)),
            jax.ShapeDtypeStruct((2, E, 32), f32),
        )(hsrc.reshape(2, E, 32), e_st, ex, rdsts).reshape(2 * E, 32)

        out2 = pl.kernel(
            _sc_scatter_out_body,
            out_type=jax.ShapeDtypeStruct((2 * NP, 32), f32),
            mesh=_sc_mesh(),
            compiler_params=_SC_PARAMS,
            scratch_types=[pltpu.VMEM((SB,), jnp.int32),
                           pltpu.VMEM((SB, 32), f32),
                           pltpu.VMEM((TF,), jnp.int32),
                           pltpu.VMEM((TF, 32), f32),
                           pltpu.VMEM((SB, 32), f32),
                           pltpu.VMEM_SHARED((NP, 32), f32)],
        )(dst, msg, z32)
        out_st = jnp.stack([out2[:N], out2[NP:NP + N]], axis=0)

        node, nb_st, nc_st = _tc(
            _node_post_body, (N // bn,),
            [pl.BlockSpec((2, bn, 32), lambda i: (0, i, 0)),
             pl.BlockSpec((bn, D), lambda i: (i, 0)),
             _full((1, D)), _full((1, D)), _full((1, D)),
             _full((D, D)), _full((D, D))],
            [pl.BlockSpec((bn, D), lambda i: (i, 0)),
             pl.BlockSpec((2, bn, 32), lambda i: (0, i, 0)),
             pl.BlockSpec((2, bn, 32), lambda i: (0, i, 0))],
            [jax.ShapeDtypeStruct((N, D), f32),
             jax.ShapeDtypeStruct((2, N, 32), f32),
             jax.ShapeDtypeStruct((2, N, 32), f32)],
        )(out_st, node, gat_bias[l].reshape(1, D), ln_g[l].reshape(1, D),
          ln_b[l].reshape(1, D), eu_W[l, D:2 * D], eu_W[l, 2 * D:])

        nbg, ncg = pl.kernel(
            _sc_gather_nbnc_body,
            out_type=[jax.ShapeDtypeStruct((2 * E, 32), f32),
                      jax.ShapeDtypeStruct((2 * E, 32), f32)],
            mesh=_sc_mesh(),
            compiler_params=_SC_PARAMS,
            scratch_types=[pltpu.VMEM((SB,), jnp.int32),
                           pltpu.VMEM((SB,), jnp.int32),
                           pltpu.VMEM((SB,), jnp.int32),
                           pltpu.VMEM((SB,), jnp.int32),
                           pltpu.VMEM((SB, 32), f32),
                           pltpu.VMEM((SB, 32), f32),
                           pltpu.VMEM((TF,), jnp.int32),
                           pltpu.VMEM((TF,), jnp.int32),
                           pltpu.VMEM((TF,), jnp.int32),
                           pltpu.VMEM((TF,), jnp.int32),
                           pltpu.VMEM((TF, 32), f32),
                           pltpu.VMEM((TF, 32), f32),
                           pltpu.SemaphoreType.DMA],
        )(src, dst, nb_st.reshape(2 * N, 32), nc_st.reshape(2 * N, 32))

        edge = _tc(
            _edge_upd_body, (E // be,),
            [pl.BlockSpec((be, D), lambda i: (i, 0)),
             pl.BlockSpec((2, be, 32), lambda i: (0, i, 0)),
             pl.BlockSpec((2, be, 32), lambda i: (0, i, 0)),
             _full((D, D)), _full((1, D))],
            pl.BlockSpec((be, D), lambda i: (i, 0)),
            jax.ShapeDtypeStruct((E, D), f32),
        )(edge, nbg.reshape(2, E, 32), ncg.reshape(2, E, 32),
          eu_W[l, :D], eu_b[l].reshape(1, D))

    return (node, edge)


# speedup vs baseline: 14.9634x; 1.3059x over previous
import functools

import jax
import jax.numpy as jnp
from jax import lax
from jax.experimental import pallas as pl
from jax.experimental.pallas import tpu as pltpu
from jax.experimental.pallas import tpu_sc as plsc

N = 50000
E = 800000
D = 64
H = 4
C = 16
L = 4
NC = 2
NS = 16
NP = 51200
NTW = NP // NS

EH = E // NC
ET_HALF = EH // NS
ET_FULL = E // NS

_SC_PARAMS = pltpu.CompilerParams(use_tc_tiling_on_sc=False)


@functools.lru_cache(maxsize=None)
def _sc_mesh():
    return plsc.VectorSubcoreMesh(core_axis_name="c", subcore_axis_name="s",
                                  num_cores=NC, num_subcores=NS)


def _enc_node_body(x_ref, wn_ref, o_ref):
    xb = x_ref[...]
    cols = []
    io = lax.broadcasted_iota(jnp.int32, (xb.shape[0], 16), 1)
    for i in range(9):
        cols.append((xb[:, i:i + 1] == io).astype(jnp.float32))
    oh = jnp.concatenate(cols, axis=1)
    o_ref[...] = jnp.dot(oh, wn_ref[...], preferred_element_type=jnp.float32)


def _enc_edge_body(a_ref, wb_ref, o_ref):
    ab = a_ref[...]
    io = lax.broadcasted_iota(jnp.int32, (ab.shape[0], 8), 1)
    cols = [(ab[:, i:i + 1] == io).astype(jnp.float32) for i in range(3)]
    oh = jnp.concatenate(cols, axis=1)
    o_ref[...] = jnp.dot(oh, wb_ref[...], preferred_element_type=jnp.float32)


def _node_pre_body(nd_ref, w_ref, asr_ref, adr_ref, hst_ref, as_ref, ad_ref):
    h = jnp.dot(nd_ref[...], w_ref[...], preferred_element_type=jnp.float32)
    asr = asr_ref[...]
    adr = adr_ref[...]
    a_s = []
    a_d = []
    for hd in range(H):
        hh = h[:, hd * C:(hd + 1) * C]
        a_s.append(jnp.sum(hh * asr[:, hd * C:(hd + 1) * C], axis=1, keepdims=True))
        a_d.append(jnp.sum(hh * adr[:, hd * C:(hd + 1) * C], axis=1, keepdims=True))
    zpad = jnp.zeros((h.shape[0], 12), dtype=jnp.float32)
    as_ref[...] = jnp.concatenate(a_s + [zpad], axis=1)
    ad_ref[...] = jnp.concatenate(a_d + [zpad], axis=1)
    hst_ref[...] = jnp.stack([h[:, :32], h[:, 32:]], axis=0)


def _edge_pre_body(ed_ref, we_ref, aer_ref, est_ref, ae_ref):
    e = jnp.dot(ed_ref[...], we_ref[...], preferred_element_type=jnp.float32)
    aer = aer_ref[...]
    a_e = []
    for hd in range(H):
        ee = e[:, hd * C:(hd + 1) * C]
        a_e.append(jnp.sum(ee * aer[:, hd * C:(hd + 1) * C], axis=1, keepdims=True))
    zpad = jnp.zeros((e.shape[0], 12), dtype=jnp.float32)
    ae_ref[...] = jnp.concatenate(a_e + [zpad], axis=1)
    est_ref[...] = jnp.stack([e[:, :32], e[:, 32:]], axis=0)


def _ex_body(gs_ref, gd_ref, ae_ref, ex_ref):
    al = gs_ref[...] + gd_ref[...] + ae_ref[...]
    al = jnp.maximum(al, 0.2 * al)
    ex_ref[...] = jnp.exp(al)


def _rden_body(p0_ref, p1_ref, o_ref):
    o_ref[...] = 1.0 / (p0_ref[...] + p1_ref[...] + 1e-16)


def _msg_body(hs_ref, es_ref, ex_ref, rd_ref, msg_ref):
    attn = ex_ref[...] * rd_ref[...]
    he = hs_ref[...] + es_ref[...]
    ones = jnp.ones((1, C), dtype=jnp.float32)
    halves = []
    for f in range(2):
        aa = jnp.concatenate(
            [attn[:, 2 * f + j:2 * f + j + 1] * ones for j in range(2)], axis=1)
        halves.append(he[f] * aa)
    msg_ref[...] = jnp.stack(halves, axis=0)


def _node_post_body(out_ref, nd_ref, bias_ref, g_ref, b_ref, wb_ref, wc_ref,
                    nn_ref, nb_ref, nc_ref):
    ob = out_ref[...]
    out = jnp.concatenate([ob[0], ob[1]], axis=1) + bias_ref[...]
    mu = jnp.mean(out, axis=1, keepdims=True)
    xc = out - mu
    var = jnp.mean(xc * xc, axis=1, keepdims=True)
    conv = g_ref[...] * xc * lax.rsqrt(var + 1e-5) + b_ref[...]
    nn = jnp.maximum(conv, 0.0) + nd_ref[...]
    nn_ref[...] = nn
    nb = jnp.dot(nn, wb_ref[...], preferred_element_type=jnp.float32)
    ncv = jnp.dot(nn, wc_ref[...], preferred_element_type=jnp.float32)
    nb_ref[...] = jnp.stack([nb[:, :32], nb[:, 32:]], axis=0)
    nc_ref[...] = jnp.stack([ncv[:, :32], ncv[:, 32:]], axis=0)


def _edge_upd_body(ed_ref, nb_ref, nc_ref, wa_ref, eb_ref, o_ref):
    nb = nb_ref[...]
    nc = nc_ref[...]
    sd64 = jnp.concatenate([nb[0] + nc[0], nb[1] + nc[1]], axis=1)
    up = jnp.dot(ed_ref[...], wa_ref[...], preferred_element_type=jnp.float32)
    o_ref[...] = jnp.maximum(ed_ref[...] + up + sd64 + eb_ref[...], 0.0)


def _full(shape):
    return pl.BlockSpec(shape, lambda i: tuple(0 for _ in shape))


SB = 128
SUP = 1024
CH = SUP // SB
NF_HALF = ET_HALF // SUP
TAIL_HALF = [SB] * 3 + [40]
NF_FULL = ET_FULL // SUP
TAIL_FULL = [SB] * 6 + [80]
SUPO = 512
CHO = SUPO // SB
NFO = ET_FULL // SUPO
TAILO = [SB] * 2 + [80]
NBW = NTW // SB


def _fire_idx(idx_hbm, base, sizes, ilist, sem):
    ds_ = []
    off = 0
    for j, sz in enumerate(sizes):
        ds_.append(pltpu.async_copy(idx_hbm.at[pl.ds(base + off, sz)],
                                    ilist[j], sem))
        off += sz
    return ds_


def _drain(ds_):
    for d in ds_:
        d.wait()


def _adjust(sizes, ilist, delta):
    for j, sz in enumerate(sizes):
        for r in range(sz // 16):
            ilist[j][pl.ds(r * 16, 16)] = ilist[j][pl.ds(r * 16, 16)] + delta


def _fire_gather(table_hbm, sizes, ilist, stage, sem):
    ds_ = []
    off = 0
    for j, sz in enumerate(sizes):
        ds_.append(pltpu.async_copy(table_hbm.at[ilist[j]],
                                    stage.at[pl.ds(off, sz)], sem))
        off += sz
    return ds_


def _fire_scatter_add(stage, sizes, ilist, acc_sp, sem):
    ds_ = []
    off = 0
    for j, sz in enumerate(sizes):
        ds_.append(pltpu.async_copy(stage.at[pl.ds(off, sz)],
                                    acc_sp.at[ilist[j]], sem, add=True))
        off += sz
    return ds_


def _sc_gather_ad_body(src_hbm, dst_hbm, ast_hbm, adt_hbm, gs_hbm, gd_hbm,
                       *scr):
    ils = list(scr[0:9])
    ild = list(scr[9:18])
    gsv, gdv, sem = scr[18], scr[19], scr[20]
    cid = lax.axis_index("c")
    sid = lax.axis_index("s")
    tile_base = cid * EH + sid * ET_HALF

    def round_(base, sizes, a_ils, a_ild):
        n = sum(sizes)
        _drain(_fire_idx(src_hbm, base, sizes, a_ils, sem)
               + _fire_idx(dst_hbm, base, sizes, a_ild, sem))
        _drain(_fire_gather(ast_hbm, sizes, a_ils, gsv, sem)
               + _fire_gather(adt_hbm, sizes, a_ild, gdv, sem))
        pltpu.sync_copy(gsv.at[pl.ds(0, n)], gs_hbm.at[pl.ds(base, n)])
        pltpu.sync_copy(gdv.at[pl.ds(0, n)], gd_hbm.at[pl.ds(base, n)])

    pl.loop(0, NF_HALF)(
        lambda k: round_(tile_base + k * SUP, [SB] * CH, ils[:8], ild[:8]))
    round_(tile_base + NF_HALF * SUP, TAIL_HALF,
           ils[:3] + [ils[8]], ild[:3] + [ild[8]])


def _sc_den_body(dst_hbm, ex_hbm, z16_hbm, den_hbm, *scr):
    ild = list(scr[0:9])
    exv, zb, sem, den_sp = scr[9], scr[10], scr[11], scr[12]
    cid = lax.axis_index("c")
    sid = lax.axis_index("s")
    w0 = sid * NTW
    pltpu.sync_copy(z16_hbm.at[pl.ds(0, SB)], zb)
    pl.loop(0, NBW)(
        lambda j: pltpu.sync_copy(zb, den_sp.at[pl.ds(w0 + j * SB, SB)]))
    plsc.subcore_barrier()
    tile_base = cid * EH + sid * ET_HALF

    def round_(base, sizes, a_ild):
        n = sum(sizes)
        ds_ = _fire_idx(dst_hbm, base, sizes, a_ild, sem)
        ds_.append(pltpu.async_copy(ex_hbm.at[pl.ds(base, n)],
                                    exv.at[pl.ds(0, n)], sem))
        _drain(ds_)
        _drain(_fire_scatter_add(exv, sizes, a_ild, den_sp, sem))

    pl.loop(0, NF_HALF)(
        lambda k: round_(tile_base + k * SUP, [SB] * CH, ild[:8]))
    round_(tile_base + NF_HALF * SUP, TAIL_HALF, ild[:3] + [ild[8]])
    plsc.subcore_barrier()

    def wb(j):
        pltpu.sync_copy(den_sp.at[pl.ds(w0 + j * SB, SB)], zb)
        pltpu.sync_copy(zb, den_hbm.at[pl.ds(cid * NP + w0 + j * SB, SB)])

    pl.loop(0, NBW)(wb)


def _sc_gather_h_body(src_hbm, dst_hbm, h_hbm, rden_hbm, hs_hbm, rd_hbm,
                      *scr):
    ils = list(scr[0:9])
    ild = list(scr[9:18])
    hv, rdv, sem = scr[18], scr[19], scr[20]
    cid = lax.axis_index("c")
    sid = lax.axis_index("s")
    tile_base = sid * ET_FULL

    def round_(base, sizes, a_ils, a_ild):
        n = sum(sizes)
        _drain(_fire_idx(src_hbm, base, sizes, a_ils, sem))
        _adjust(sizes, a_ils, cid * N)
        _drain(_fire_gather(h_hbm, sizes, a_ils, hv, sem))
        pltpu.sync_copy(hv.at[pl.ds(0, n)],
                        hs_hbm.at[pl.ds(cid * E + base, n)])

        @pl.when(cid == 0)
        def _():
            _drain(_fire_idx(dst_hbm, base, sizes, a_ild, sem))
            _drain(_fire_gather(rden_hbm, sizes, a_ild, rdv, sem))
            pltpu.sync_copy(rdv.at[pl.ds(0, n)], rd_hbm.at[pl.ds(base, n)])

    pl.loop(0, NF_FULL)(
        lambda k: round_(tile_base + k * SUP, [SB] * CH, ils[:8], ild[:8]))
    round_(tile_base + NF_FULL * SUP, TAIL_FULL,
           ils[:6] + [ils[8]], ild[:6] + [ild[8]])


def _sc_scatter_out_body(dst_hbm, msg_hbm, z32_hbm, out_hbm, *scr):
    ild = list(scr[0:5])
    mv, zb, sem, out_sp = scr[5], scr[6], scr[7], scr[8]
    cid = lax.axis_index("c")
    sid = lax.axis_index("s")
    w0 = sid * NTW
    pltpu.sync_copy(z32_hbm.at[pl.ds(0, SB)], zb)
    pl.loop(0, NBW)(
        lambda j: pltpu.sync_copy(zb, out_sp.at[pl.ds(w0 + j * SB, SB)]))
    plsc.subcore_barrier()
    tile_base = sid * ET_FULL

    def round_(base, sizes, a_ild):
        n = sum(sizes)
        ds_ = _fire_idx(dst_hbm, base, sizes, a_ild, sem)
        ds_.append(pltpu.async_copy(msg_hbm.at[pl.ds(cid * E + base, n)],
                                    mv.at[pl.ds(0, n)], sem))
        _drain(ds_)
        _drain(_fire_scatter_add(mv, sizes, a_ild, out_sp, sem))

    pl.loop(0, NFO)(
        lambda k: round_(tile_base + k * SUPO, [SB] * CHO, ild[:4]))
    round_(tile_base + NFO * SUPO, TAILO, ild[:2] + [ild[4]])
    plsc.subcore_barrier()

    def wb(j):
        pltpu.sync_copy(out_sp.at[pl.ds(w0 + j * SB, SB)], zb)
        pltpu.sync_copy(zb, out_hbm.at[pl.ds(cid * NP + w0 + j * SB, SB)])

    pl.loop(0, NBW)(wb)


def _sc_gather_nbnc_body(src_hbm, dst_hbm, nb_hbm, nc_hbm, nbg_hbm, ncg_hbm,
                         *scr):
    ils = list(scr[0:9])
    ild = list(scr[9:18])
    bv, cv, sem = scr[18], scr[19], scr[20]
    cid = lax.axis_index("c")
    sid = lax.axis_index("s")
    tile_base = sid * ET_FULL

    def round_(base, sizes, a_ils, a_ild):
        n = sum(sizes)
        _drain(_fire_idx(src_hbm, base, sizes, a_ils, sem)
               + _fire_idx(dst_hbm, base, sizes, a_ild, sem))
        _adjust(sizes, a_ils, cid * N)
        _adjust(sizes, a_ild, cid * N)
        _drain(_fire_gather(nb_hbm, sizes, a_ils, bv, sem)
               + _fire_gather(nc_hbm, sizes, a_ild, cv, sem))
        pltpu.sync_copy(bv.at[pl.ds(0, n)],
                        nbg_hbm.at[pl.ds(cid * E + base, n)])
        pltpu.sync_copy(cv.at[pl.ds(0, n)],
                        ncg_hbm.at[pl.ds(cid * E + base, n)])

    pl.loop(0, NF_FULL)(
        lambda k: round_(tile_base + k * SUP, [SB] * CH, ils[:8], ild[:8]))
    round_(tile_base + NF_FULL * SUP, TAIL_FULL,
           ils[:6] + [ils[8]], ild[:6] + [ild[8]])


def _idx_scratch(tail, n=8):
    return [pltpu.VMEM((SB,), jnp.int32) for _ in range(n)] + \
        [pltpu.VMEM((tail,), jnp.int32)]


def _tc(body, grid, in_specs, out_specs, out_shapes):
    return pl.pallas_call(body, grid=grid, in_specs=in_specs,
                          out_specs=out_specs, out_shape=out_shapes)


def kernel(x, edge_index, edge_attr, atom_emb, bond_emb, gat_W, att_src,
           att_dst, gat_We, att_edge, gat_bias, ln_g, ln_b, eu_W, eu_b):
    f32 = jnp.float32
    src = edge_index[0]
    dst = edge_index[1]

    wn = atom_emb[:, :16, :].reshape(144, D)
    bn = 2000
    node = _tc(_enc_node_body, (N // bn,),
               [pl.BlockSpec((bn, 9), lambda i: (i, 0)), _full((144, D))],
               pl.BlockSpec((bn, D), lambda i: (i, 0)),
               jax.ShapeDtypeStruct((N, D), f32))(x, wn)

    wb = bond_emb[:, :8, :].reshape(24, D)
    be = 8000
    edge = _tc(_enc_edge_body, (E // be,),
               [pl.BlockSpec((be, 3), lambda i: (i, 0)), _full((24, D))],
               pl.BlockSpec((be, D), lambda i: (i, 0)),
               jax.ShapeDtypeStruct((E, D), f32))(edge_attr, wb)

    z16 = jnp.zeros((NP, 16), f32)
    z32 = jnp.zeros((NP, 32), f32)

    for l in range(L):
        h_st, as_t, ad_t = _tc(
            _node_pre_body, (N // bn,),
            [pl.BlockSpec((bn, D), lambda i: (i, 0)), _full((D, D)),
             _full((1, D)), _full((1, D))],
            [pl.BlockSpec((2, bn, 32), lambda i: (0, i, 0)),
             pl.BlockSpec((bn, 16), lambda i: (i, 0)),
             pl.BlockSpec((bn, 16), lambda i: (i, 0))],
            [jax.ShapeDtypeStruct((2, N, 32), f32),
             jax.ShapeDtypeStruct((N, 16), f32),
             jax.ShapeDtypeStruct((N, 16), f32)],
        )(node, gat_W[l], att_src[l].reshape(1, D), att_dst[l].reshape(1, D))
        h_flat = h_st.reshape(2 * N, 32)

        e_st, ae_t = _tc(
            _edge_pre_body, (E // be,),
            [pl.BlockSpec((be, D), lambda i: (i, 0)), _full((D, D)),
             _full((1, D))],
            [pl.BlockSpec((2, be, 32), lambda i: (0, i, 0)),
             pl.BlockSpec((be, 16), lambda i: (i, 0))],
            [jax.ShapeDtypeStruct((2, E, 32), f32),
             jax.ShapeDtypeStruct((E, 16), f32)],
        )(edge, gat_We[l], att_edge[l].reshape(1, D))

        gs, gd = pl.kernel(
            _sc_gather_ad_body,
            out_type=[jax.ShapeDtypeStruct((E, 16), f32),
                      jax.ShapeDtypeStruct((E, 16), f32)],
            mesh=_sc_mesh(),
            compiler_params=_SC_PARAMS,
            scratch_types=_idx_scratch(40) + _idx_scratch(40) +
                          [pltpu.VMEM((SUP, 16), f32),
                           pltpu.VMEM((SUP, 16), f32),
                           pltpu.SemaphoreType.DMA],
        )(src, dst, as_t, ad_t)

        rr = E // 8
        bx = 10000
        ex = _tc(_ex_body, (rr // bx,),
                 [pl.BlockSpec((bx, 128), lambda i: (i, 0))] * 3,
                 pl.BlockSpec((bx, 128), lambda i: (i, 0)),
                 jax.ShapeDtypeStruct((rr, 128), f32))(
            gs.reshape(rr, 128), gd.reshape(rr, 128),
            ae_t.reshape(rr, 128)).reshape(E, 16)

        den2 = pl.kernel(
            _sc_den_body,
            out_type=jax.ShapeDtypeStruct((2 * NP, 16), f32),
            mesh=_sc_mesh(),
            compiler_params=_SC_PARAMS,
            scratch_types=_idx_scratch(40) +
                          [pltpu.VMEM((SUP, 16), f32),
                           pltpu.VMEM((SB, 16), f32),
                           pltpu.SemaphoreType.DMA,
                           pltpu.VMEM_SHARED((NP, 16), f32)],
        )(dst, ex, z16)

        brd = 6250
        rden = _tc(_rden_body, (1,),
                   [pl.BlockSpec((brd, 128), lambda i: (i, 0))] * 2,
                   pl.BlockSpec((brd, 128), lambda i: (i, 0)),
                   jax.ShapeDtypeStruct((brd, 128), f32))(
            den2[:N].reshape(brd, 128),
            den2[NP:NP + N].reshape(brd, 128)).reshape(N, 16)

        hsrc, rdsts = pl.kernel(
            _sc_gather_h_body,
            out_type=[jax.ShapeDtypeStruct((2 * E, 32), f32),
                      jax.ShapeDtypeStruct((E, 16), f32)],
            mesh=_sc_mesh(),
            compiler_params=_SC_PARAMS,
            scratch_types=_idx_scratch(80) + _idx_scratch(80) +
                          [pltpu.VMEM((SUP, 32), f32),
                           pltpu.VMEM((SUP, 16), f32),
                           pltpu.SemaphoreType.DMA],
        )(src, dst, h_flat, rden)

        bm = 2000
        msg = _tc(
            _msg_body, (E // bm,),
            [pl.BlockSpec((2, bm, 32), lambda i: (0, i, 0)),
             pl.BlockSpec((2, bm, 32), lambda i: (0, i, 0)),
             pl.BlockSpec((bm, 16), lambda i: (i, 0)),
             pl.BlockSpec((bm, 16), lambda i: (i, 0))],
            pl.BlockSpec((2, bm, 32), lambda i: (0, i, 0)),
            jax.ShapeDtypeStruct((2, E, 32), f32),
        )(hsrc.reshape(2, E, 32), e_st, ex, rdsts).reshape(2 * E, 32)

        out2 = pl.kernel(
            _sc_scatter_out_body,
            out_type=jax.ShapeDtypeStruct((2 * NP, 32), f32),
            mesh=_sc_mesh(),
            compiler_params=_SC_PARAMS,
            scratch_types=_idx_scratch(80, n=4) +
                          [pltpu.VMEM((SUPO, 32), f32),
                           pltpu.VMEM((SB, 32), f32),
                           pltpu.SemaphoreType.DMA,
                           pltpu.VMEM_SHARED((NP, 32), f32)],
        )(dst, msg, z32)
        out_st = jnp.stack([out2[:N], out2[NP:NP + N]], axis=0)

        node, nb_st, nc_st = _tc(
            _node_post_body, (N // bn,),
            [pl.BlockSpec((2, bn, 32), lambda i: (0, i, 0)),
             pl.BlockSpec((bn, D), lambda i: (i, 0)),
             _full((1, D)), _full((1, D)), _full((1, D)),
             _full((D, D)), _full((D, D))],
            [pl.BlockSpec((bn, D), lambda i: (i, 0)),
             pl.BlockSpec((2, bn, 32), lambda i: (0, i, 0)),
             pl.BlockSpec((2, bn, 32), lambda i: (0, i, 0))],
            [jax.ShapeDtypeStruct((N, D), f32),
             jax.ShapeDtypeStruct((2, N, 32), f32),
             jax.ShapeDtypeStruct((2, N, 32), f32)],
        )(out_st, node, gat_bias[l].reshape(1, D), ln_g[l].reshape(1, D),
          ln_b[l].reshape(1, D), eu_W[l, D:2 * D], eu_W[l, 2 * D:])

        nbg, ncg = pl.kernel(
            _sc_gather_nbnc_body,
            out_type=[jax.ShapeDtypeStruct((2 * E, 32), f32),
                      jax.ShapeDtypeStruct((2 * E, 32), f32)],
            mesh=_sc_mesh(),
            compiler_params=_SC_PARAMS,
            scratch_types=_idx_scratch(80) + _idx_scratch(80) +
                          [pltpu.VMEM((SUP, 32), f32),
                           pltpu.VMEM((SUP, 32), f32),
                           pltpu.SemaphoreType.DMA],
        )(src, dst, nb_st.reshape(2 * N, 32), nc_st.reshape(2 * N, 32))

        edge = _tc(
            _edge_upd_body, (E // be,),
            [pl.BlockSpec((be, D), lambda i: (i, 0)),
             pl.BlockSpec((2, be, 32), lambda i: (0, i, 0)),
             pl.BlockSpec((2, be, 32), lambda i: (0, i, 0)),
             _full((D, D)), _full((1, D))],
            pl.BlockSpec((be, D), lambda i: (i, 0)),
            jax.ShapeDtypeStruct((E, D), f32),
        )(edge, nbg.reshape(2, E, 32), ncg.reshape(2, E, 32),
          eu_W[l, :D], eu_b[l].reshape(1, D))

    return (node, edge)


# speedup vs baseline: 15.0342x; 1.0047x over previous
import functools

import jax
import jax.numpy as jnp
from jax import lax
from jax.experimental import pallas as pl
from jax.experimental.pallas import tpu as pltpu
from jax.experimental.pallas import tpu_sc as plsc

N = 50000
E = 800000
D = 64
H = 4
C = 16
L = 4
NC = 2
NS = 16
NP = 51200
NTW = NP // NS

EH = E // NC
ET_HALF = EH // NS
ET_FULL = E // NS

_SC_PARAMS = pltpu.CompilerParams(use_tc_tiling_on_sc=False)


@functools.lru_cache(maxsize=None)
def _sc_mesh():
    return plsc.VectorSubcoreMesh(core_axis_name="c", subcore_axis_name="s",
                                  num_cores=NC, num_subcores=NS)


def _enc_node_body(x_ref, wn_ref, o_ref):
    xb = x_ref[...]
    cols = []
    io = lax.broadcasted_iota(jnp.int32, (xb.shape[0], 16), 1)
    for i in range(9):
        cols.append((xb[:, i:i + 1] == io).astype(jnp.float32))
    oh = jnp.concatenate(cols, axis=1)
    o_ref[...] = jnp.dot(oh, wn_ref[...], preferred_element_type=jnp.float32)


def _enc_edge_body(a_ref, wb_ref, o_ref):
    ab = a_ref[...]
    io = lax.broadcasted_iota(jnp.int32, (ab.shape[0], 8), 1)
    cols = [(ab[:, i:i + 1] == io).astype(jnp.float32) for i in range(3)]
    oh = jnp.concatenate(cols, axis=1)
    o_ref[...] = jnp.dot(oh, wb_ref[...], preferred_element_type=jnp.float32)


def _node_pre_body(nd_ref, w_ref, asr_ref, adr_ref, hst_ref, as_ref, ad_ref):
    h = jnp.dot(nd_ref[...], w_ref[...], preferred_element_type=jnp.float32)
    asr = asr_ref[...]
    adr = adr_ref[...]
    a_s = []
    a_d = []
    for hd in range(H):
        hh = h[:, hd * C:(hd + 1) * C]
        a_s.append(jnp.sum(hh * asr[:, hd * C:(hd + 1) * C], axis=1, keepdims=True))
        a_d.append(jnp.sum(hh * adr[:, hd * C:(hd + 1) * C], axis=1, keepdims=True))
    zpad = jnp.zeros((h.shape[0], 12), dtype=jnp.float32)
    as_ref[...] = jnp.concatenate(a_s + [zpad], axis=1)
    ad_ref[...] = jnp.concatenate(a_d + [zpad], axis=1)
    hst_ref[...] = jnp.stack([h[:, :32], h[:, 32:]], axis=0)


def _edge_pre_body(ed_ref, we_ref, aer_ref, est_ref, ae_ref):
    e = jnp.dot(ed_ref[...], we_ref[...], preferred_element_type=jnp.float32)
    aer = aer_ref[...]
    a_e = []
    for hd in range(H):
        ee = e[:, hd * C:(hd + 1) * C]
        a_e.append(jnp.sum(ee * aer[:, hd * C:(hd + 1) * C], axis=1, keepdims=True))
    zpad = jnp.zeros((e.shape[0], 12), dtype=jnp.float32)
    ae_ref[...] = jnp.concatenate(a_e + [zpad], axis=1)
    est_ref[...] = jnp.stack([e[:, :32], e[:, 32:]], axis=0)


def _ex_body(gs_ref, gd_ref, ae_ref, ex_ref):
    al = gs_ref[...] + gd_ref[...] + ae_ref[...]
    al = jnp.maximum(al, 0.2 * al)
    ex_ref[...] = jnp.exp(al)


def _rden_body(p0_ref, p1_ref, o_ref):
    o_ref[...] = 1.0 / (p0_ref[...] + p1_ref[...] + 1e-16)


def _msg_body(hs_ref, es_ref, ex_ref, rd_ref, msg_ref):
    attn = ex_ref[...] * rd_ref[...]
    he = hs_ref[...] + es_ref[...]
    ones = jnp.ones((1, C), dtype=jnp.float32)
    halves = []
    for f in range(2):
        aa = jnp.concatenate(
            [attn[:, 2 * f + j:2 * f + j + 1] * ones for j in range(2)], axis=1)
        halves.append(he[f] * aa)
    msg_ref[...] = jnp.stack(halves, axis=0)


def _node_post_body(out_ref, nd_ref, bias_ref, g_ref, b_ref, wb_ref, wc_ref,
                    nn_ref, nb_ref, nc_ref):
    ob = out_ref[...]
    out = jnp.concatenate([ob[0], ob[1]], axis=1) + bias_ref[...]
    mu = jnp.mean(out, axis=1, keepdims=True)
    xc = out - mu
    var = jnp.mean(xc * xc, axis=1, keepdims=True)
    conv = g_ref[...] * xc * lax.rsqrt(var + 1e-5) + b_ref[...]
    nn = jnp.maximum(conv, 0.0) + nd_ref[...]
    nn_ref[...] = nn
    nb = jnp.dot(nn, wb_ref[...], preferred_element_type=jnp.float32)
    ncv = jnp.dot(nn, wc_ref[...], preferred_element_type=jnp.float32)
    nb_ref[...] = jnp.stack([nb[:, :32], nb[:, 32:]], axis=0)
    nc_ref[...] = jnp.stack([ncv[:, :32], ncv[:, 32:]], axis=0)


def _edge_upd_body(ed_ref, nb_ref, nc_ref, wa_ref, eb_ref, o_ref):
    nb = nb_ref[...]
    nc = nc_ref[...]
    sd64 = jnp.concatenate([nb[0] + nc[0], nb[1] + nc[1]], axis=1)
    up = jnp.dot(ed_ref[...], wa_ref[...], preferred_element_type=jnp.float32)
    o_ref[...] = jnp.maximum(ed_ref[...] + up + sd64 + eb_ref[...], 0.0)


def _full(shape):
    return pl.BlockSpec(shape, lambda i: tuple(0 for _ in shape))


SB = 128
SUP = 1024
CH = SUP // SB
NF_HALF = ET_HALF // SUP
TAIL_HALF = [SB] * 3 + [40]
NF_FULL = ET_FULL // SUP
TAIL_FULL = [SB] * 6 + [80]
SUPO = 512
CHO = SUPO // SB
NFO = ET_FULL // SUPO
TAILO = [SB] * 2 + [80]
NBW = NTW // SB


def _fire_idx(idx_hbm, base, sizes, ilist, sem):
    ds_ = []
    off = 0
    for j, sz in enumerate(sizes):
        ds_.append(pltpu.async_copy(idx_hbm.at[pl.ds(base + off, sz)],
                                    ilist[j], sem))
        off += sz
    return ds_


def _drain(ds_):
    for d in ds_:
        d.wait()


def _adjust(sizes, ilist, delta):
    for j, sz in enumerate(sizes):
        for r in range(sz // 16):
            ilist[j][pl.ds(r * 16, 16)] = ilist[j][pl.ds(r * 16, 16)] + delta


def _fire_gather(table_hbm, sizes, ilist, stage, sem):
    ds_ = []
    off = 0
    for j, sz in enumerate(sizes):
        ds_.append(pltpu.async_copy(table_hbm.at[ilist[j]],
                                    stage.at[pl.ds(off, sz)], sem))
        off += sz
    return ds_


def _fire_scatter_add(stage, sizes, ilist, acc_sp, sem):
    ds_ = []
    off = 0
    for j, sz in enumerate(sizes):
        ds_.append(pltpu.async_copy(stage.at[pl.ds(off, sz)],
                                    acc_sp.at[ilist[j]], sem, add=True))
        off += sz
    return ds_


def _sc_gather_ad_body(src_hbm, dst_hbm, ast_hbm, adt_hbm, gs_hbm, gd_hbm,
                       *scr):
    ils = list(scr[0:9])
    ild = list(scr[9:18])
    gsv, gdv, sem = scr[18], scr[19], scr[20]
    cid = lax.axis_index("c")
    sid = lax.axis_index("s")
    tile_base = cid * EH + sid * ET_HALF

    def round_(base, sizes, a_ils, a_ild):
        n = sum(sizes)
        _drain(_fire_idx(src_hbm, base, sizes, a_ils, sem)
               + _fire_idx(dst_hbm, base, sizes, a_ild, sem))
        _drain(_fire_gather(ast_hbm, sizes, a_ils, gsv, sem)
               + _fire_gather(adt_hbm, sizes, a_ild, gdv, sem))
        pltpu.sync_copy(gsv.at[pl.ds(0, n)], gs_hbm.at[pl.ds(base, n)])
        pltpu.sync_copy(gdv.at[pl.ds(0, n)], gd_hbm.at[pl.ds(base, n)])

    pl.loop(0, NF_HALF)(
        lambda k: round_(tile_base + k * SUP, [SB] * CH, ils[:8], ild[:8]))
    round_(tile_base + NF_HALF * SUP, TAIL_HALF,
           ils[:3] + [ils[8]], ild[:3] + [ild[8]])


def _sc_den_body(dst_hbm, ex_hbm, z16_hbm, den_hbm, *scr):
    ild = list(scr[0:9])
    exv, zb, sem, den_sp = scr[9], scr[10], scr[11], scr[12]
    cid = lax.axis_index("c")
    sid = lax.axis_index("s")
    w0 = sid * NTW
    pltpu.sync_copy(z16_hbm.at[pl.ds(0, SB)], zb)
    pl.loop(0, NBW)(
        lambda j: pltpu.sync_copy(zb, den_sp.at[pl.ds(w0 + j * SB, SB)]))
    plsc.subcore_barrier()
    tile_base = cid * EH + sid * ET_HALF

    def round_(base, sizes, a_ild):
        n = sum(sizes)
        ds_ = _fire_idx(dst_hbm, base, sizes, a_ild, sem)
        ds_.append(pltpu.async_copy(ex_hbm.at[pl.ds(base, n)],
                                    exv.at[pl.ds(0, n)], sem))
        _drain(ds_)
        _drain(_fire_scatter_add(exv, sizes, a_ild, den_sp, sem))

    pl.loop(0, NF_HALF)(
        lambda k: round_(tile_base + k * SUP, [SB] * CH, ild[:8]))
    round_(tile_base + NF_HALF * SUP, TAIL_HALF, ild[:3] + [ild[8]])
    plsc.subcore_barrier()

    def wb(j):
        pltpu.sync_copy(den_sp.at[pl.ds(w0 + j * SB, SB)], zb)
        pltpu.sync_copy(zb, den_hbm.at[pl.ds(cid * NP + w0 + j * SB, SB)])

    pl.loop(0, NBW)(wb)


def _sc_gather_h_body(src_hbm, dst_hbm, h_hbm, rden_hbm, hs_hbm, rd_hbm,
                      *scr):
    ils = list(scr[0:9])
    ild = list(scr[9:18])
    hv, rdv, sem = scr[18], scr[19], scr[20]
    cid = lax.axis_index("c")
    sid = lax.axis_index("s")
    tile_base = sid * ET_FULL

    def round_(base, sizes, a_ils, a_ild):
        n = sum(sizes)
        _drain(_fire_idx(src_hbm, base, sizes, a_ils, sem))
        _adjust(sizes, a_ils, cid * N)
        _drain(_fire_gather(h_hbm, sizes, a_ils, hv, sem))
        pltpu.sync_copy(hv.at[pl.ds(0, n)],
                        hs_hbm.at[pl.ds(cid * E + base, n)])

        @pl.when(cid == 0)
        def _():
            _drain(_fire_idx(dst_hbm, base, sizes, a_ild, sem))
            _drain(_fire_gather(rden_hbm, sizes, a_ild, rdv, sem))
            pltpu.sync_copy(rdv.at[pl.ds(0, n)], rd_hbm.at[pl.ds(base, n)])

    pl.loop(0, NF_FULL)(
        lambda k: round_(tile_base + k * SUP, [SB] * CH, ils[:8], ild[:8]))
    round_(tile_base + NF_FULL * SUP, TAIL_FULL,
           ils[:6] + [ils[8]], ild[:6] + [ild[8]])


def _sc_scatter_out_body(dst_hbm, msg_hbm, z32_hbm, out_hbm, *scr):
    ild = list(scr[0:5])
    mv, zb, sem, out_sp = scr[5], scr[6], scr[7], scr[8]
    cid = lax.axis_index("c")
    sid = lax.axis_index("s")
    w0 = sid * NTW
    pltpu.sync_copy(z32_hbm.at[pl.ds(0, SB)], zb)
    pl.loop(0, NBW)(
        lambda j: pltpu.sync_copy(zb, out_sp.at[pl.ds(w0 + j * SB, SB)]))
    plsc.subcore_barrier()
    tile_base = sid * ET_FULL

    def round_(base, sizes, a_ild):
        n = sum(sizes)
        ds_ = _fire_idx(dst_hbm, base, sizes, a_ild, sem)
        ds_.append(pltpu.async_copy(msg_hbm.at[pl.ds(cid * E + base, n)],
                                    mv.at[pl.ds(0, n)], sem))
        _drain(ds_)
        _drain(_fire_scatter_add(mv, sizes, a_ild, out_sp, sem))

    pl.loop(0, NFO)(
        lambda k: round_(tile_base + k * SUPO, [SB] * CHO, ild[:4]))
    round_(tile_base + NFO * SUPO, TAILO, ild[:2] + [ild[4]])
    plsc.subcore_barrier()

    def wb(j):
        pltpu.sync_copy(out_sp.at[pl.ds(w0 + j * SB, SB)], zb)
        pltpu.sync_copy(zb, out_hbm.at[pl.ds(cid * NP + w0 + j * SB, SB)])

    pl.loop(0, NBW)(wb)


def _sc_gather_nbnc_body(src_hbm, dst_hbm, nb_hbm, nc_hbm, nbg_hbm, ncg_hbm,
                         *scr):
    ils = list(scr[0:9])
    ild = list(scr[9:18])
    bv, cv, sem = scr[18], scr[19], scr[20]
    cid = lax.axis_index("c")
    sid = lax.axis_index("s")
    tile_base = sid * ET_FULL

    def round_(base, sizes, a_ils, a_ild):
        n = sum(sizes)
        _drain(_fire_idx(src_hbm, base, sizes, a_ils, sem)
               + _fire_idx(dst_hbm, base, sizes, a_ild, sem))
        _adjust(sizes, a_ils, cid * N)
        _adjust(sizes, a_ild, cid * N)
        _drain(_fire_gather(nb_hbm, sizes, a_ils, bv, sem)
               + _fire_gather(nc_hbm, sizes, a_ild, cv, sem))
        pltpu.sync_copy(bv.at[pl.ds(0, n)],
                        nbg_hbm.at[pl.ds(cid * E + base, n)])
        pltpu.sync_copy(cv.at[pl.ds(0, n)],
                        ncg_hbm.at[pl.ds(cid * E + base, n)])

    pl.loop(0, NF_FULL)(
        lambda k: round_(tile_base + k * SUP, [SB] * CH, ils[:8], ild[:8]))
    round_(tile_base + NF_FULL * SUP, TAIL_FULL,
           ils[:6] + [ils[8]], ild[:6] + [ild[8]])


def _idx_scratch(tail, n=8):
    return [pltpu.VMEM((SB,), jnp.int32) for _ in range(n)] + \
        [pltpu.VMEM((tail,), jnp.int32)]


def _tc(body, grid, in_specs, out_specs, out_shapes):
    return pl.pallas_call(body, grid=grid, in_specs=in_specs,
                          out_specs=out_specs, out_shape=out_shapes)


def kernel(x, edge_index, edge_attr, atom_emb, bond_emb, gat_W, att_src,
           att_dst, gat_We, att_edge, gat_bias, ln_g, ln_b, eu_W, eu_b):
    f32 = jnp.float32
    src = edge_index[0]
    dst = edge_index[1]

    wn = atom_emb[:, :16, :].reshape(144, D)
    bn = 2000
    node = _tc(_enc_node_body, (N // bn,),
               [pl.BlockSpec((bn, 9), lambda i: (i, 0)), _full((144, D))],
               pl.BlockSpec((bn, D), lambda i: (i, 0)),
               jax.ShapeDtypeStruct((N, D), f32))(x, wn)

    wb = bond_emb[:, :8, :].reshape(24, D)
    be = 8000
    edge = _tc(_enc_edge_body, (E // be,),
               [pl.BlockSpec((be, 3), lambda i: (i, 0)), _full((24, D))],
               pl.BlockSpec((be, D), lambda i: (i, 0)),
               jax.ShapeDtypeStruct((E, D), f32))(edge_attr, wb)

    z16 = jnp.zeros((NP, 16), f32)
    z32 = jnp.zeros((NP, 32), f32)

    for l in range(L):
        h_st, as_t, ad_t = _tc(
            _node_pre_body, (N // bn,),
            [pl.BlockSpec((bn, D), lambda i: (i, 0)), _full((D, D)),
             _full((1, D)), _full((1, D))],
            [pl.BlockSpec((2, bn, 32), lambda i: (0, i, 0)),
             pl.BlockSpec((bn, 16), lambda i: (i, 0)),
             pl.BlockSpec((bn, 16), lambda i: (i, 0))],
            [jax.ShapeDtypeStruct((2, N, 32), f32),
             jax.ShapeDtypeStruct((N, 16), f32),
             jax.ShapeDtypeStruct((N, 16), f32)],
        )(node, gat_W[l], att_src[l].reshape(1, D), att_dst[l].reshape(1, D))
        h_flat = h_st.reshape(2 * N, 32)

        e_st, ae_t = _tc(
            _edge_pre_body, (E // be,),
            [pl.BlockSpec((be, D), lambda i: (i, 0)), _full((D, D)),
             _full((1, D))],
            [pl.BlockSpec((2, be, 32), lambda i: (0, i, 0)),
             pl.BlockSpec((be, 16), lambda i: (i, 0))],
            [jax.ShapeDtypeStruct((2, E, 32), f32),
             jax.ShapeDtypeStruct((E, 16), f32)],
        )(edge, gat_We[l], att_edge[l].reshape(1, D))

        gs, gd = pl.kernel(
            _sc_gather_ad_body,
            out_type=[jax.ShapeDtypeStruct((E, 16), f32),
                      jax.ShapeDtypeStruct((E, 16), f32)],
            mesh=_sc_mesh(),
            compiler_params=_SC_PARAMS,
            scratch_types=_idx_scratch(40) + _idx_scratch(40) +
                          [pltpu.VMEM((SUP, 16), f32),
                           pltpu.VMEM((SUP, 16), f32),
                           pltpu.SemaphoreType.DMA],
        )(src, dst, as_t, ad_t)

        rr = E // 8
        bx = 10000
        ex = _tc(_ex_body, (rr // bx,),
                 [pl.BlockSpec((bx, 128), lambda i: (i, 0))] * 3,
                 pl.BlockSpec((bx, 128), lambda i: (i, 0)),
                 jax.ShapeDtypeStruct((rr, 128), f32))(
            gs.reshape(rr, 128), gd.reshape(rr, 128),
            ae_t.reshape(rr, 128)).reshape(E, 16)

        den2 = pl.kernel(
            _sc_den_body,
            out_type=jax.ShapeDtypeStruct((2 * NP, 16), f32),
            mesh=_sc_mesh(),
            compiler_params=_SC_PARAMS,
            scratch_types=_idx_scratch(40) +
                          [pltpu.VMEM((SUP, 16), f32),
                           pltpu.VMEM((SB, 16), f32),
                           pltpu.SemaphoreType.DMA,
                           pltpu.VMEM_SHARED((NP, 16), f32)],
        )(dst, ex, z16)

        brd = 6250
        rden = _tc(_rden_body, (1,),
                   [pl.BlockSpec((brd, 128), lambda i: (i, 0))] * 2,
                   pl.BlockSpec((brd, 128), lambda i: (i, 0)),
                   jax.ShapeDtypeStruct((brd, 128), f32))(
            den2[:N].reshape(brd, 128),
            den2[NP:NP + N].reshape(brd, 128)).reshape(N, 16)

        hsrc, rdsts = pl.kernel(
            _sc_gather_h_body,
            out_type=[jax.ShapeDtypeStruct((2 * E, 32), f32),
                      jax.ShapeDtypeStruct((E, 16), f32)],
            mesh=_sc_mesh(),
            compiler_params=_SC_PARAMS,
            scratch_types=_idx_scratch(80) + _idx_scratch(80) +
                          [pltpu.VMEM((SUP, 32), f32),
                           pltpu.VMEM((SUP, 16), f32),
                           pltpu.SemaphoreType.DMA],
        )(src, dst, h_flat, rden)

        bm = 4000
        msg = _tc(
            _msg_body, (E // bm,),
            [pl.BlockSpec((2, bm, 32), lambda i: (0, i, 0)),
             pl.BlockSpec((2, bm, 32), lambda i: (0, i, 0)),
             pl.BlockSpec((bm, 16), lambda i: (i, 0)),
             pl.BlockSpec((bm, 16), lambda i: (i, 0))],
            pl.BlockSpec((2, bm, 32), lambda i: (0, i, 0)),
            jax.ShapeDtypeStruct((2, E, 32), f32),
        )(hsrc.reshape(2, E, 32), e_st, ex, rdsts).reshape(2 * E, 32)

        out2 = pl.kernel(
            _sc_scatter_out_body,
            out_type=jax.ShapeDtypeStruct((2 * NP, 32), f32),
            mesh=_sc_mesh(),
            compiler_params=_SC_PARAMS,
            scratch_types=_idx_scratch(80, n=4) +
                          [pltpu.VMEM((SUPO, 32), f32),
                           pltpu.VMEM((SB, 32), f32),
                           pltpu.SemaphoreType.DMA,
                           pltpu.VMEM_SHARED((NP, 32), f32)],
        )(dst, msg, z32)
        out_st = jnp.stack([out2[:N], out2[NP:NP + N]], axis=0)

        node, nb_st, nc_st = _tc(
            _node_post_body, (N // bn,),
            [pl.BlockSpec((2, bn, 32), lambda i: (0, i, 0)),
             pl.BlockSpec((bn, D), lambda i: (i, 0)),
             _full((1, D)), _full((1, D)), _full((1, D)),
             _full((D, D)), _full((D, D))],
            [pl.BlockSpec((bn, D), lambda i: (i, 0)),
             pl.BlockSpec((2, bn, 32), lambda i: (0, i, 0)),
             pl.BlockSpec((2, bn, 32), lambda i: (0, i, 0))],
            [jax.ShapeDtypeStruct((N, D), f32),
             jax.ShapeDtypeStruct((2, N, 32), f32),
             jax.ShapeDtypeStruct((2, N, 32), f32)],
        )(out_st, node, gat_bias[l].reshape(1, D), ln_g[l].reshape(1, D),
          ln_b[l].reshape(1, D), eu_W[l, D:2 * D], eu_W[l, 2 * D:])

        nbg, ncg = pl.kernel(
            _sc_gather_nbnc_body,
            out_type=[jax.ShapeDtypeStruct((2 * E, 32), f32),
                      jax.ShapeDtypeStruct((2 * E, 32), f32)],
            mesh=_sc_mesh(),
            compiler_params=_SC_PARAMS,
            scratch_types=_idx_scratch(80) + _idx_scratch(80) +
                          [pltpu.VMEM((SUP, 32), f32),
                           pltpu.VMEM((SUP, 32), f32),
                           pltpu.SemaphoreType.DMA],
        )(src, dst, nb_st.reshape(2 * N, 32), nc_st.reshape(2 * N, 32))

        edge = _tc(
            _edge_upd_body, (E // be,),
            [pl.BlockSpec((be, D), lambda i: (i, 0)),
             pl.BlockSpec((2, be, 32), lambda i: (0, i, 0)),
             pl.BlockSpec((2, be, 32), lambda i: (0, i, 0)),
             _full((D, D)), _full((1, D))],
            pl.BlockSpec((be, D), lambda i: (i, 0)),
            jax.ShapeDtypeStruct((E, D), f32),
        )(edge, nbg.reshape(2, E, 32), ncg.reshape(2, E, 32),
          eu_W[l, :D], eu_b[l].reshape(1, D))

    return (node, edge)
